# baseline (speedup 1.0000x reference)
import functools

import jax
import jax.numpy as jnp
from jax import lax
from jax.experimental import pallas as pl
from jax.experimental.pallas import tpu as pltpu
from jax.experimental.pallas import tpu_sc as plsc

B, D1, D2 = 4, 4096, 2048
NC, NS, L = 2, 16, 16
NW = NC * NS
CPB = 128
NBLK = D2 // CPB
NG = CPB // L
TPW = (B * NBLK) // NW
CHUNK = 256
NCHUNK = D1 // CHUNK
UNROLL = 4


def _argmin_sc(x):
    mesh = plsc.VectorSubcoreMesh(core_axis_name="c", subcore_axis_name="s")

    @functools.partial(
        pl.kernel,
        mesh=mesh,
        out_type=jax.ShapeDtypeStruct((B * D2,), jnp.int32),
        scratch_types=[
            pltpu.VMEM((CHUNK, CPB), jnp.float32),
            pltpu.VMEM((CHUNK, CPB), jnp.float32),
            pltpu.VMEM((TPW * CPB,), jnp.int32),
            pltpu.SemaphoreType.DMA,
            pltpu.SemaphoreType.DMA,
        ],
    )
    def k(x_hbm, out_hbm, buf0, buf1, idx_v, sem0, sem1):
        wid = lax.axis_index("s") * NC + lax.axis_index("c")
        bufs = (buf0, buf1)
        sems = (sem0, sem1)
        total = TPW * NCHUNK

        def start(i):
            t, ch = divmod(i, NCHUNK)
            task = wid * TPW + t
            b = task // NBLK
            c0 = (task % NBLK) * CPB
            return pltpu.async_copy(
                x_hbm.at[b, pl.ds(ch * CHUNK, CHUNK), pl.ds(c0, CPB)],
                bufs[i % 2], sems[i % 2])

        handles = [None] * total
        handles[0] = start(0)
        for t in range(TPW):
            mins = tuple(jnp.full((L,), jnp.inf, jnp.float32) for _ in range(NG))
            idxs = tuple(jnp.zeros((L,), jnp.int32) for _ in range(NG))
            for ch in range(NCHUNK):
                i = t * NCHUNK + ch
                if i + 1 < total:
                    handles[i + 1] = start(i + 1)
                handles[i].wait()
                buf = bufs[i % 2]
                base = ch * CHUNK

                def body(it, carry, buf=buf, base=base):
                    mins, idxs = carry
                    mins, idxs = list(mins), list(idxs)
                    r0 = it * UNROLL
                    for u in range(UNROLL):
                        rvec = jnp.full((L,), base + r0 + u, jnp.int32)
                        for j in range(NG):
                            v = buf[r0 + u, pl.ds(j * L, L)]
                            m = v < mins[j]
                            mins[j] = jnp.where(m, v, mins[j])
                            idxs[j] = jnp.where(m, rvec, idxs[j])
                    return tuple(mins), tuple(idxs)

                mins, idxs = lax.fori_loop(0, CHUNK // UNROLL, body,
                                           (mins, idxs))
            for j in range(NG):
                idx_v[pl.ds(t * CPB + j * L, L)] = idxs[j]
        for t in range(TPW):
            task = wid * TPW + t
            b = task // NBLK
            c0 = (task % NBLK) * CPB
            pltpu.sync_copy(idx_v.at[pl.ds(t * CPB, CPB)],
                            out_hbm.at[pl.ds(b * D2 + c0, CPB)])

    return k(x)


def kernel(x):
    return _argmin_sc(x).reshape(B, D2)


# speedup vs baseline: 1.3931x; 1.3931x over previous
import functools

import jax
import jax.numpy as jnp
from jax import lax
from jax.experimental import pallas as pl
from jax.experimental.pallas import tpu as pltpu
from jax.experimental.pallas import tpu_sc as plsc

B, D1, D2 = 4, 4096, 2048
NC, NS, L = 2, 16, 16
NW = NC * NS
CPB = 128
NBLK = D2 // CPB
NG = CPB // L
TPW = (B * NBLK) // NW
CHUNK = 256
NCHUNK = D1 // CHUNK
UNROLL = 4


def _argmin_sc(x):
    mesh = plsc.VectorSubcoreMesh(core_axis_name="c", subcore_axis_name="s")

    @functools.partial(
        pl.kernel,
        mesh=mesh,
        out_type=jax.ShapeDtypeStruct((B * D2,), jnp.int32),
        scratch_types=[
            pltpu.VMEM((CHUNK, CPB), jnp.float32),
            pltpu.VMEM((CHUNK, CPB), jnp.float32),
            pltpu.VMEM((TPW * CPB,), jnp.int32),
            pltpu.SemaphoreType.DMA,
            pltpu.SemaphoreType.DMA,
        ],
    )
    def k(x_hbm, out_hbm, buf0, buf1, idx_v, sem0, sem1):
        wid = lax.axis_index("s") * NC + lax.axis_index("c")
        bufs = (buf0, buf1)
        sems = (sem0, sem1)
        total = TPW * NCHUNK

        def start(i):
            t, ch = divmod(i, NCHUNK)
            task = wid * TPW + t
            b = task // NBLK
            c0 = (task % NBLK) * CPB
            return pltpu.async_copy(
                x_hbm.at[b, pl.ds(ch * CHUNK, CHUNK), pl.ds(c0, CPB)],
                bufs[i % 2], sems[i % 2])

        handles = [None] * total
        handles[0] = start(0)
        for t in range(TPW):
            mins = tuple(jnp.full((L,), jnp.inf, jnp.float32) for _ in range(NG))
            idxs = tuple(jnp.zeros((L,), jnp.int32) for _ in range(NG))
            for ch in range(NCHUNK):
                i = t * NCHUNK + ch
                if i + 1 < total:
                    handles[i + 1] = start(i + 1)
                handles[i].wait()
                buf = bufs[i % 2]
                base = ch * CHUNK

                def body(it, carry, buf=buf, base=base):
                    mins, idxs = carry
                    mins, idxs = list(mins), list(idxs)
                    r0 = it * UNROLL
                    for u in range(UNROLL):
                        rvec = jnp.full((L,), base + r0 + u, jnp.int32)
                        for j in range(NG):
                            v = buf[r0 + u, pl.ds(j * L, L)]
                            m = v < mins[j]
                            mins[j] = jnp.where(m, v, mins[j])
                            idxs[j] = jnp.where(m, rvec, idxs[j])
                    return tuple(mins), tuple(idxs)

                mins, idxs = body(0, (mins, idxs))
            for j in range(NG):
                idx_v[pl.ds(t * CPB + j * L, L)] = idxs[j]
        for t in range(TPW):
            task = wid * TPW + t
            b = task // NBLK
            c0 = (task % NBLK) * CPB
            pltpu.sync_copy(idx_v.at[pl.ds(t * CPB, CPB)],
                            out_hbm.at[pl.ds(b * D2 + c0, CPB)])

    return k(x)


def kernel(x):
    return _argmin_sc(x).reshape(B, D2)


# speedup vs baseline: 2.1277x; 1.5274x over previous
import jax
import jax.numpy as jnp
from jax import lax
from jax.experimental import pallas as pl

B, D1, D2 = 4, 4096, 2048
CB = 128


def _argmin_tc(x):
    ncb = D2 // CB

    def body(x_ref, o_ref):
        for b in range(B):
            xb = x_ref[b]
            minv = jnp.min(xb, axis=0, keepdims=True)
            iota = lax.broadcasted_iota(jnp.int32, (D1, CB), 0)
            idx = jnp.min(jnp.where(xb == minv, iota, jnp.int32(D1)), axis=0)
            o_ref[b, :] = idx

    return pl.pallas_call(
        body,
        grid=(ncb,),
        in_specs=[pl.BlockSpec((B, D1, CB), lambda c: (0, 0, c))],
        out_specs=pl.BlockSpec((B, CB), lambda c: (0, c)),
        out_shape=jax.ShapeDtypeStruct((B, D2), jnp.int32),
    )(x)


def kernel(x):
    return _argmin_tc(x)
